# baseline (speedup 1.0000x reference)
import math

import jax
import jax.numpy as jnp
from jax.experimental import pallas as pl
from jax.experimental.pallas import tpu as pltpu

D = 1024
H = 16
T = 2048
BLK = 64
NB = T // BLK
KC = int(math.ceil(0.5 * NB))
DH = D // H
SCALE = 1.0 / math.sqrt(DH)
ROWS = 256

_HI = jax.lax.Precision.HIGHEST
_BF = jnp.bfloat16


def _dot_t(a, w):
    return jax.lax.dot_general(a.astype(_BF), w.astype(_BF),
                               (((1,), (1,)), ((), ())),
                               preferred_element_type=jnp.float32)


def _ln_qkv_kernel(x_ref, g_ref, b_ref, wq_ref, bq_ref, wk_ref, bk_ref,
                   wv_ref, bv_ref, q_ref, k_ref, v_ref):
    x = x_ref[...]
    mu = jnp.mean(x, axis=-1, keepdims=True)
    xc = x - mu
    var = jnp.mean(xc * xc, axis=-1, keepdims=True)
    xn = xc / jnp.sqrt(var + 1e-5) * g_ref[...] + b_ref[...]
    q_ref[...] = _dot_t(xn, wq_ref[...]) + bq_ref[...]
    k_ref[...] = _dot_t(xn, wk_ref[...]) + bk_ref[...]
    v_ref[...] = _dot_t(xn, wv_ref[...]) + bv_ref[...]


def _select_kernel(q_ref, k_ref, idx_ref, cnt_ref):
    q = q_ref[...]
    k = k_ref[...]
    qm = jnp.mean(q.reshape(NB, BLK, D), axis=1)
    km = jnp.mean(k.reshape(NB, BLK, D), axis=1)
    qh = qm.reshape(NB, H, DH).transpose(1, 0, 2)
    kh = km.reshape(NB, H, DH).transpose(1, 0, 2)
    sim = jax.lax.dot_general(qh.astype(_BF), kh.astype(_BF),
                              (((2,), (2,)), ((0,), (0,))),
                              preferred_element_type=jnp.float32)
    gt = (sim[:, :, None, :] > sim[:, :, :, None]).astype(jnp.float32)
    cntg = jnp.sum(gt, axis=-1)
    ii = jax.lax.broadcasted_iota(jnp.int32, (H, NB, NB), 1)
    jj = jax.lax.broadcasted_iota(jnp.int32, (H, NB, NB), 2)
    keep = (cntg < float(KC)) | (jj == ii)
    keep = keep & (jj <= ii)
    kf = keep.astype(jnp.float32)
    cnt_ref[...] = jnp.sum(keep.astype(jnp.int32), axis=-1)
    a_io = jax.lax.broadcasted_iota(jnp.int32, (NB, NB), 0)
    b_io = jax.lax.broadcasted_iota(jnp.int32, (NB, NB), 1)
    upper = (a_io <= b_io).astype(jnp.float32)
    pos = jax.lax.dot_general(kf, upper, (((2,), (0,)), ((), ())),
                              preferred_element_type=jnp.float32,
                              precision=_HI) - 1.0
    pos_i = pos.astype(jnp.int32)
    s_io = jax.lax.broadcasted_iota(jnp.int32, (H, NB, NB, NB), 3)
    onehot = ((pos_i[..., None] == s_io) & keep[..., None]).astype(jnp.int32)
    idx_ref[...] = jnp.sum(onehot * jj[..., None], axis=2)


def _attn_kernel(idx_ref, cnt_ref, q_ref, k_ref, v_ref, o_ref):
    h = pl.program_id(0)
    i = pl.program_id(1)
    n = cnt_ref[h * NB + i]
    q = (q_ref[0] * SCALE).astype(_BF)
    row = jax.lax.broadcasted_iota(jnp.int32, (BLK, BLK), 0)
    col = jax.lax.broadcasted_iota(jnp.int32, (BLK, BLK), 1)
    tril = row >= col

    def body(s, carry):
        m, l, acc = carry
        j = idx_ref[h * NB * NB + i * NB + s]
        kb = k_ref[0, pl.ds(j * BLK, BLK), :]
        vb = v_ref[0, pl.ds(j * BLK, BLK), :]
        sc = jax.lax.dot_general(q, kb.astype(_BF), (((1,), (1,)), ((), ())),
                                 preferred_element_type=jnp.float32)
        sc = jnp.where(jnp.logical_or(j < i, tril), sc, -1e30)
        mnew = jnp.maximum(m, jnp.max(sc, axis=1, keepdims=True))
        alpha = jnp.exp(m - mnew)
        p = jnp.exp(sc - mnew)
        l2 = l * alpha + jnp.sum(p, axis=1, keepdims=True)
        acc2 = acc * alpha + jax.lax.dot_general(
            p.astype(_BF), vb.astype(_BF), (((1,), (0,)), ((), ())),
            preferred_element_type=jnp.float32)
        return mnew, l2, acc2

    m0 = jnp.full((BLK, 1), -1e30, jnp.float32)
    l0 = jnp.zeros((BLK, 1), jnp.float32)
    a0 = jnp.zeros((BLK, DH), jnp.float32)
    m, l, acc = jax.lax.fori_loop(0, n, body, (m0, l0, a0))
    o_ref[0] = acc / l


def _outproj_kernel(c_ref, wo_ref, bo_ref, o_ref):
    o_ref[...] = _dot_t(c_ref[...], wo_ref[...]) + bo_ref[...]


def kernel(inputs, ln_g, ln_b, Wq, bq, Wk, bk, Wv, bv, Wo, bo):
    x = inputs.reshape(T, D)
    g2 = ln_g.reshape(1, D)
    b2 = ln_b.reshape(1, D)
    bq2 = bq.reshape(1, D)
    bk2 = bk.reshape(1, D)
    bv2 = bv.reshape(1, D)
    bo2 = bo.reshape(1, D)

    full = pl.BlockSpec((D, D), lambda r: (0, 0))
    vec = pl.BlockSpec((1, D), lambda r: (0, 0))
    rows = pl.BlockSpec((ROWS, D), lambda r: (r, 0))
    q, k, v = pl.pallas_call(
        _ln_qkv_kernel,
        grid=(T // ROWS,),
        in_specs=[rows, vec, vec, full, vec, full, vec, full, vec],
        out_specs=[rows, rows, rows],
        out_shape=[jax.ShapeDtypeStruct((T, D), jnp.float32)] * 3,
    )(x, g2, b2, Wq, bq2, Wk, bk2, Wv, bv2)

    idx, cnt = pl.pallas_call(
        _select_kernel,
        grid=(1,),
        in_specs=[pl.BlockSpec((T, D), lambda r: (0, 0))] * 2,
        out_specs=[pl.BlockSpec((H, NB, NB), lambda r: (0, 0, 0)),
                   pl.BlockSpec((H, NB), lambda r: (0, 0))],
        out_shape=[jax.ShapeDtypeStruct((H, NB, NB), jnp.int32),
                   jax.ShapeDtypeStruct((H, NB), jnp.int32)],
    )(q, k)

    qh = q.reshape(T, H, DH).transpose(1, 0, 2)
    kh = k.reshape(T, H, DH).transpose(1, 0, 2)
    vh = v.reshape(T, H, DH).transpose(1, 0, 2)

    grid_spec = pltpu.PrefetchScalarGridSpec(
        num_scalar_prefetch=2,
        grid=(H, NB),
        in_specs=[
            pl.BlockSpec((1, BLK, DH), lambda h, i, *_: (h, i, 0)),
            pl.BlockSpec((1, T, DH), lambda h, i, *_: (h, 0, 0)),
            pl.BlockSpec((1, T, DH), lambda h, i, *_: (h, 0, 0)),
        ],
        out_specs=pl.BlockSpec((1, BLK, DH), lambda h, i, *_: (h, i, 0)),
    )
    ctx = pl.pallas_call(
        _attn_kernel,
        grid_spec=grid_spec,
        out_shape=jax.ShapeDtypeStruct((H, T, DH), jnp.float32),
    )(idx.reshape(-1), cnt.reshape(-1), qh, kh, vh)
    ctx2 = ctx.transpose(1, 0, 2).reshape(T, D)

    out = pl.pallas_call(
        _outproj_kernel,
        grid=(T // ROWS,),
        in_specs=[rows, full, vec],
        out_specs=rows,
        out_shape=jax.ShapeDtypeStruct((T, D), jnp.float32),
    )(ctx2, Wo, bo2)

    return out.reshape(1, T, D)


# speedup vs baseline: 4.0746x; 4.0746x over previous
import math

import jax
import jax.numpy as jnp
from jax.experimental import pallas as pl
from jax.experimental.pallas import tpu as pltpu

D = 1024
H = 16
T = 2048
BLK = 64
NB = T // BLK
KC = int(math.ceil(0.5 * NB))
DH = D // H
SCALE = 1.0 / math.sqrt(DH)
ROWS = 256

_HI = jax.lax.Precision.HIGHEST
_BF = jnp.bfloat16


def _dot_t(a, w):
    return jax.lax.dot_general(a.astype(_BF), w.astype(_BF),
                               (((1,), (1,)), ((), ())),
                               preferred_element_type=jnp.float32)


def _ln_qkv_kernel(x_ref, g_ref, b_ref, wq_ref, bq_ref, wk_ref, bk_ref,
                   wv_ref, bv_ref, q_ref, k_ref, v_ref):
    x = x_ref[...]
    mu = jnp.mean(x, axis=-1, keepdims=True)
    xc = x - mu
    var = jnp.mean(xc * xc, axis=-1, keepdims=True)
    xn = xc / jnp.sqrt(var + 1e-5) * g_ref[...] + b_ref[...]
    q_ref[...] = _dot_t(xn, wq_ref[...]) + bq_ref[...]
    k_ref[...] = _dot_t(xn, wk_ref[...]) + bk_ref[...]
    v_ref[...] = _dot_t(xn, wv_ref[...]) + bv_ref[...]


def _select_kernel(q_ref, k_ref, mask_ref):
    q = q_ref[...]
    k = k_ref[...]
    qm = jnp.mean(q.reshape(NB, BLK, D), axis=1)
    km = jnp.mean(k.reshape(NB, BLK, D), axis=1)
    qh = qm.reshape(NB, H, DH).transpose(1, 0, 2)
    kh = km.reshape(NB, H, DH).transpose(1, 0, 2)
    sim = jax.lax.dot_general(qh.astype(_BF), kh.astype(_BF),
                              (((2,), (2,)), ((0,), (0,))),
                              preferred_element_type=jnp.float32)
    gt = (sim[:, :, None, :] > sim[:, :, :, None]).astype(jnp.float32)
    cntg = jnp.sum(gt, axis=-1)
    ii = jax.lax.broadcasted_iota(jnp.int32, (H, NB, NB), 1)
    jj = jax.lax.broadcasted_iota(jnp.int32, (H, NB, NB), 2)
    keep = (cntg < float(KC)) | (jj == ii)
    drop = 1.0 - keep.astype(jnp.float32)
    j_io = jax.lax.broadcasted_iota(jnp.int32, (NB, T), 0)
    c_io = jax.lax.broadcasted_iota(jnp.int32, (NB, T), 1)
    expander = (c_io // BLK == j_io).astype(jnp.float32)
    mask_ref[...] = jax.lax.dot_general(
        drop * (-1e30), expander, (((2,), (0,)), ((), ())),
        preferred_element_type=jnp.float32, precision=_HI)


QT = 256
GB = QT // BLK


def _attn_kernel(q_ref, k_ref, v_ref, mask_ref, o_ref):
    i = pl.program_id(1)
    q = (q_ref[0] * SCALE).astype(_BF)
    row = jax.lax.broadcasted_iota(jnp.int32, (QT, QT), 0)
    col = jax.lax.broadcasted_iota(jnp.int32, (QT, QT), 1)
    causal_add = jnp.where(row >= col, 0.0, -1e30)

    def body(g, carry):
        m, l, acc = carry
        kb = k_ref[0, pl.ds(g * QT, QT), :].astype(_BF)
        vb = v_ref[0, pl.ds(g * QT, QT), :].astype(_BF)
        sc = jax.lax.dot_general(q, kb, (((1,), (1,)), ((), ())),
                                 preferred_element_type=jnp.float32)
        m4 = mask_ref[0, 0, :, pl.ds(g * QT, QT)]
        madd = jnp.concatenate(
            [jnp.broadcast_to(m4[a:a + 1, :], (BLK, QT)) for a in range(GB)],
            axis=0)
        sc = sc + madd + jnp.where(g == i, causal_add, 0.0)
        mnew = jnp.maximum(m, jnp.max(sc, axis=1, keepdims=True))
        alpha = jnp.exp(m - mnew)
        p = jnp.exp(sc - mnew)
        l2 = l * alpha + jnp.sum(p, axis=1, keepdims=True)
        acc2 = acc * alpha + jax.lax.dot_general(
            p.astype(_BF), vb, (((1,), (0,)), ((), ())),
            preferred_element_type=jnp.float32)
        return mnew, l2, acc2

    m0 = jnp.full((QT, 1), -1e30, jnp.float32)
    l0 = jnp.zeros((QT, 1), jnp.float32)
    a0 = jnp.zeros((QT, DH), jnp.float32)
    m, l, acc = jax.lax.fori_loop(0, i + 1, body, (m0, l0, a0))
    o_ref[0] = acc / l


def _outproj_kernel(c_ref, wo_ref, bo_ref, o_ref):
    o_ref[...] = _dot_t(c_ref[...], wo_ref[...]) + bo_ref[...]


def kernel(inputs, ln_g, ln_b, Wq, bq, Wk, bk, Wv, bv, Wo, bo):
    x = inputs.reshape(T, D)
    g2 = ln_g.reshape(1, D)
    b2 = ln_b.reshape(1, D)
    bq2 = bq.reshape(1, D)
    bk2 = bk.reshape(1, D)
    bv2 = bv.reshape(1, D)
    bo2 = bo.reshape(1, D)

    full = pl.BlockSpec((D, D), lambda r: (0, 0))
    vec = pl.BlockSpec((1, D), lambda r: (0, 0))
    rows = pl.BlockSpec((ROWS, D), lambda r: (r, 0))
    q, k, v = pl.pallas_call(
        _ln_qkv_kernel,
        grid=(T // ROWS,),
        in_specs=[rows, vec, vec, full, vec, full, vec, full, vec],
        out_specs=[rows, rows, rows],
        out_shape=[jax.ShapeDtypeStruct((T, D), jnp.float32)] * 3,
    )(x, g2, b2, Wq, bq2, Wk, bk2, Wv, bv2)

    amask = pl.pallas_call(
        _select_kernel,
        grid=(1,),
        in_specs=[pl.BlockSpec((T, D), lambda r: (0, 0))] * 2,
        out_specs=pl.BlockSpec((H, NB, T), lambda r: (0, 0, 0)),
        out_shape=jax.ShapeDtypeStruct((H, NB, T), jnp.float32),
    )(q, k)
    amask4 = amask.reshape(H, NB // GB, GB, T)

    qh = q.reshape(T, H, DH).transpose(1, 0, 2)
    kh = k.reshape(T, H, DH).transpose(1, 0, 2)
    vh = v.reshape(T, H, DH).transpose(1, 0, 2)

    ctx = pl.pallas_call(
        _attn_kernel,
        grid=(H, T // QT),
        in_specs=[
            pl.BlockSpec((1, QT, DH), lambda h, i: (h, i, 0)),
            pl.BlockSpec((1, T, DH), lambda h, i: (h, 0, 0)),
            pl.BlockSpec((1, T, DH), lambda h, i: (h, 0, 0)),
            pl.BlockSpec((1, 1, GB, T), lambda h, i: (h, i, 0, 0)),
        ],
        out_specs=pl.BlockSpec((1, QT, DH), lambda h, i: (h, i, 0)),
        out_shape=jax.ShapeDtypeStruct((H, T, DH), jnp.float32),
    )(qh, kh, vh, amask4)
    ctx2 = ctx.transpose(1, 0, 2).reshape(T, D)

    out = pl.pallas_call(
        _outproj_kernel,
        grid=(T // ROWS,),
        in_specs=[rows, full, vec],
        out_specs=rows,
        out_shape=jax.ShapeDtypeStruct((T, D), jnp.float32),
    )(ctx2, Wo, bo2)

    return out.reshape(1, T, D)


# speedup vs baseline: 4.3603x; 1.0701x over previous
import math

import jax
import jax.numpy as jnp
from jax.experimental import pallas as pl
from jax.experimental.pallas import tpu as pltpu

D = 1024
H = 16
T = 2048
BLK = 64
NB = T // BLK
KC = int(math.ceil(0.5 * NB))
DH = D // H
SCALE = 1.0 / math.sqrt(DH)
ROWS = 256
QT = 256
GB = QT // BLK

_BF = jnp.bfloat16


def _qkv_sel_kernel(xn_ref, wq_ref, bq_ref, wk_ref, bk_ref,
                    wv_ref, bv_ref, q_ref, k_ref, v_ref, mask_ref):
    xn = xn_ref[...]
    w = jnp.concatenate([wq_ref[0], wk_ref[0], wv_ref[0]],
                        axis=0).astype(_BF)
    qkv = jax.lax.dot_general(xn, w, (((1,), (1,)), ((), ())),
                              preferred_element_type=jnp.float32)
    qh = qkv[:, :DH] + bq_ref[0]
    kh = qkv[:, DH:2 * DH] + bk_ref[0]
    vh = qkv[:, 2 * DH:] + bv_ref[0]
    q_ref[0] = qh
    k_ref[0] = kh
    v_ref[0] = vh
    qm = jnp.mean(qh.reshape(NB, BLK, DH), axis=1)
    km = jnp.mean(kh.reshape(NB, BLK, DH), axis=1)
    sim = jax.lax.dot_general(qm.astype(_BF), km.astype(_BF),
                              (((1,), (1,)), ((), ())),
                              preferred_element_type=jnp.float32)
    gt = (sim[:, None, :] > sim[:, :, None]).astype(jnp.float32)
    cntg = jnp.sum(gt, axis=-1)
    ii = jax.lax.broadcasted_iota(jnp.int32, (NB, NB), 0)
    jj = jax.lax.broadcasted_iota(jnp.int32, (NB, NB), 1)
    keep = (cntg < float(KC)) | (jj == ii)
    drop = 1.0 - keep.astype(jnp.float32)
    j_io = jax.lax.broadcasted_iota(jnp.int32, (NB, T), 0)
    c_io = jax.lax.broadcasted_iota(jnp.int32, (NB, T), 1)
    expander = (c_io // BLK == j_io).astype(_BF)
    mask_ref[0] = jax.lax.dot_general(
        (drop * (-1e30)).astype(_BF), expander, (((1,), (0,)), ((), ())),
        preferred_element_type=jnp.float32)


def _attn_kernel(q_ref, k_ref, v_ref, mask_ref, o_ref):
    i = pl.program_id(1)
    q = (q_ref[0] * SCALE).astype(_BF)

    def _tile(g, causal_add):
        kb = k_ref[0, pl.ds(g * QT, QT), :].astype(_BF)
        vb = v_ref[0, pl.ds(g * QT, QT), :].astype(_BF)
        sc = jax.lax.dot_general(q, kb, (((1,), (1,)), ((), ())),
                                 preferred_element_type=jnp.float32)
        m4 = mask_ref[0, 0, :, pl.ds(g * QT, QT)]
        madd = jnp.concatenate(
            [jnp.broadcast_to(m4[a:a + 1, :], (BLK, QT)) for a in range(GB)],
            axis=0)
        sc = sc + madd
        if causal_add is not None:
            sc = sc + causal_add
        return sc, vb

    def _update(sc, vb, m, l, acc):
        mnew = jnp.maximum(m, jnp.max(sc, axis=1, keepdims=True))
        alpha = jnp.exp(m - mnew)
        p = jnp.exp(sc - mnew)
        l2 = l * alpha + jnp.sum(p, axis=1, keepdims=True)
        acc2 = acc * alpha + jax.lax.dot_general(
            p.astype(_BF), vb, (((1,), (0,)), ((), ())),
            preferred_element_type=jnp.float32)
        return mnew, l2, acc2

    def body(g, carry):
        m, l, acc = carry
        sc, vb = _tile(g, None)
        return _update(sc, vb, m, l, acc)

    m0 = jnp.full((QT, 1), -1e30, jnp.float32)
    l0 = jnp.zeros((QT, 1), jnp.float32)
    a0 = jnp.zeros((QT, DH), jnp.float32)
    m, l, acc = jax.lax.fori_loop(0, i, body, (m0, l0, a0))
    row = jax.lax.broadcasted_iota(jnp.int32, (QT, QT), 0)
    col = jax.lax.broadcasted_iota(jnp.int32, (QT, QT), 1)
    causal_add = jnp.where(row >= col, 0.0, -1e30)
    sc, vb = _tile(i, causal_add)
    m, l, acc = _update(sc, vb, m, l, acc)
    o_ref[0] = acc / l


def _outproj_kernel(c_ref, wo_ref, bo_ref, o_ref):
    o_ref[...] = jax.lax.dot_general(
        c_ref[...].astype(_BF), wo_ref[...].astype(_BF),
        (((1,), (1,)), ((), ())),
        preferred_element_type=jnp.float32) + bo_ref[...]


def kernel(inputs, ln_g, ln_b, Wq, bq, Wk, bk, Wv, bv, Wo, bo):
    x = inputs.reshape(T, D)
    mu = jnp.mean(x, axis=-1, keepdims=True)
    var = jnp.mean((x - mu) ** 2, axis=-1, keepdims=True)
    xn = ((x - mu) / jnp.sqrt(var + 1e-5) * ln_g.reshape(1, D)
          + ln_b.reshape(1, D)).astype(_BF)
    wq3 = Wq.reshape(H, DH, D)
    wk3 = Wk.reshape(H, DH, D)
    wv3 = Wv.reshape(H, DH, D)
    bq3 = bq.reshape(H, 1, DH)
    bk3 = bk.reshape(H, 1, DH)
    bv3 = bv.reshape(H, 1, DH)
    bo2 = bo.reshape(1, D)

    fullx = pl.BlockSpec((T, D), lambda h: (0, 0))
    whead = pl.BlockSpec((1, DH, D), lambda h: (h, 0, 0))
    bhead = pl.BlockSpec((1, 1, DH), lambda h: (h, 0, 0))
    ohead = pl.BlockSpec((1, T, DH), lambda h: (h, 0, 0))
    qh, kh, vh, amask = pl.pallas_call(
        _qkv_sel_kernel,
        grid=(H,),
        in_specs=[fullx, whead, bhead, whead, bhead, whead, bhead],
        out_specs=[ohead, ohead, ohead,
                   pl.BlockSpec((1, NB, T), lambda h: (h, 0, 0))],
        out_shape=[jax.ShapeDtypeStruct((H, T, DH), jnp.float32)] * 3 +
                  [jax.ShapeDtypeStruct((H, NB, T), jnp.float32)],
    )(xn, wq3, bq3, wk3, bk3, wv3, bv3)
    amask4 = amask.reshape(H, NB // GB, GB, T)

    ctx = pl.pallas_call(
        _attn_kernel,
        grid=(H, T // QT),
        in_specs=[
            pl.BlockSpec((1, QT, DH), lambda h, i: (h, i, 0)),
            pl.BlockSpec((1, T, DH), lambda h, i: (h, 0, 0)),
            pl.BlockSpec((1, T, DH), lambda h, i: (h, 0, 0)),
            pl.BlockSpec((1, 1, GB, T), lambda h, i: (h, i, 0, 0)),
        ],
        out_specs=pl.BlockSpec((1, QT, DH), lambda h, i: (h, i, 0)),
        out_shape=jax.ShapeDtypeStruct((H, T, DH), jnp.float32),
    )(qh, kh, vh, amask4)
    ctx2 = ctx.transpose(1, 0, 2).reshape(T, D)

    rows = pl.BlockSpec((ROWS, D), lambda r: (r, 0))
    out = pl.pallas_call(
        _outproj_kernel,
        grid=(T // ROWS,),
        in_specs=[rows, pl.BlockSpec((D, D), lambda r: (0, 0)),
                  pl.BlockSpec((1, D), lambda r: (0, 0))],
        out_specs=rows,
        out_shape=jax.ShapeDtypeStruct((T, D), jnp.float32),
    )(ctx2, Wo, bo2)

    return out.reshape(1, T, D)


# speedup vs baseline: 7.4191x; 1.7015x over previous
import math

import jax
import jax.numpy as jnp
from jax.experimental import pallas as pl
from jax.experimental.pallas import tpu as pltpu

D = 1024
H = 16
T = 2048
BLK = 64
NB = T // BLK
KC = int(math.ceil(0.5 * NB))
DH = D // H
SCALE = 1.0 / math.sqrt(DH)
ROWS = 256
QT = 512
GB = QT // BLK

_BF = jnp.bfloat16


def _qkv_sel_kernel(xn_ref, wq_ref, bq_ref, wk_ref, bk_ref,
                    wv_ref, bv_ref, q_ref, k_ref, v_ref, mask_ref):
    xn = xn_ref[...]
    w = jnp.concatenate([wq_ref[0], wk_ref[0], wv_ref[0]],
                        axis=0).astype(_BF)
    qkv = jax.lax.dot_general(xn, w, (((1,), (1,)), ((), ())),
                              preferred_element_type=jnp.float32)
    qh = qkv[:, :DH] + bq_ref[0]
    kh = qkv[:, DH:2 * DH] + bk_ref[0]
    vh = qkv[:, 2 * DH:] + bv_ref[0]
    q_ref[0] = qh
    k_ref[0] = kh
    v_ref[0] = vh
    qm = jnp.mean(qh.reshape(NB, BLK, DH), axis=1)
    km = jnp.mean(kh.reshape(NB, BLK, DH), axis=1)
    sim = jax.lax.dot_general(qm.astype(_BF), km.astype(_BF),
                              (((1,), (1,)), ((), ())),
                              preferred_element_type=jnp.float32)
    gt = (sim[:, None, :] > sim[:, :, None]).astype(jnp.float32)
    cntg = jnp.sum(gt, axis=-1)
    ii = jax.lax.broadcasted_iota(jnp.int32, (NB, NB), 0)
    jj = jax.lax.broadcasted_iota(jnp.int32, (NB, NB), 1)
    keep = (cntg < float(KC)) | (jj == ii)
    drop = 1.0 - keep.astype(jnp.float32)
    j_io = jax.lax.broadcasted_iota(jnp.int32, (NB, T), 0)
    c_io = jax.lax.broadcasted_iota(jnp.int32, (NB, T), 1)
    expander = (c_io // BLK == j_io).astype(_BF)
    mask_ref[0] = jax.lax.dot_general(
        (drop * (-1e30)).astype(_BF), expander, (((1,), (0,)), ((), ())),
        preferred_element_type=jnp.float32)


def _attn_kernel(q_ref, k_ref, v_ref, mask_ref, o_ref):
    i = pl.program_id(1)
    q = (q_ref[0] * SCALE).astype(_BF)

    def _tile(g, causal_add):
        kb = k_ref[0, pl.ds(g * QT, QT), :].astype(_BF)
        vb = v_ref[0, pl.ds(g * QT, QT), :].astype(_BF)
        sc = jax.lax.dot_general(q, kb, (((1,), (1,)), ((), ())),
                                 preferred_element_type=jnp.float32)
        m4 = mask_ref[0, 0, :, pl.ds(g * QT, QT)]
        sc = (sc.reshape(GB, BLK, QT) + m4[:, None, :]).reshape(QT, QT)
        if causal_add is not None:
            sc = sc + causal_add
        return sc, vb

    def _update(sc, vb, m, l, acc):
        mnew = jnp.maximum(m, jnp.max(sc, axis=1, keepdims=True))
        alpha = jnp.exp(m - mnew)
        p = jnp.exp(sc - mnew)
        l2 = l * alpha + jnp.sum(p, axis=1, keepdims=True)
        acc2 = acc * alpha + jax.lax.dot_general(
            p.astype(_BF), vb, (((1,), (0,)), ((), ())),
            preferred_element_type=jnp.float32)
        return mnew, l2, acc2

    def body(g, carry):
        m, l, acc = carry
        sc, vb = _tile(g, None)
        return _update(sc, vb, m, l, acc)

    m0 = jnp.full((QT, 1), -1e30, jnp.float32)
    l0 = jnp.zeros((QT, 1), jnp.float32)
    a0 = jnp.zeros((QT, DH), jnp.float32)
    m, l, acc = jax.lax.fori_loop(0, i, body, (m0, l0, a0))
    row = jax.lax.broadcasted_iota(jnp.int32, (QT, QT), 0)
    col = jax.lax.broadcasted_iota(jnp.int32, (QT, QT), 1)
    causal_add = jnp.where(row >= col, 0.0, -1e30)
    sc, vb = _tile(i, causal_add)
    m, l, acc = _update(sc, vb, m, l, acc)
    o_ref[0] = acc / l


def _outproj_kernel(c_ref, wo_ref, bo_ref, o_ref):
    o_ref[...] = jax.lax.dot_general(
        c_ref[...].astype(_BF), wo_ref[...].astype(_BF),
        (((1,), (1,)), ((), ())),
        preferred_element_type=jnp.float32) + bo_ref[...]


def kernel(inputs, ln_g, ln_b, Wq, bq, Wk, bk, Wv, bv, Wo, bo):
    x = inputs.reshape(T, D)
    mu = jnp.mean(x, axis=-1, keepdims=True)
    var = jnp.mean((x - mu) ** 2, axis=-1, keepdims=True)
    xn = ((x - mu) / jnp.sqrt(var + 1e-5) * ln_g.reshape(1, D)
          + ln_b.reshape(1, D)).astype(_BF)
    wq3 = Wq.reshape(H, DH, D)
    wk3 = Wk.reshape(H, DH, D)
    wv3 = Wv.reshape(H, DH, D)
    bq3 = bq.reshape(H, 1, DH)
    bk3 = bk.reshape(H, 1, DH)
    bv3 = bv.reshape(H, 1, DH)
    bo2 = bo.reshape(1, D)

    fullx = pl.BlockSpec((T, D), lambda h: (0, 0))
    whead = pl.BlockSpec((1, DH, D), lambda h: (h, 0, 0))
    bhead = pl.BlockSpec((1, 1, DH), lambda h: (h, 0, 0))
    ohead = pl.BlockSpec((1, T, DH), lambda h: (h, 0, 0))
    qh, kh, vh, amask = pl.pallas_call(
        _qkv_sel_kernel,
        grid=(H,),
        in_specs=[fullx, whead, bhead, whead, bhead, whead, bhead],
        out_specs=[ohead, ohead, ohead,
                   pl.BlockSpec((1, NB, T), lambda h: (h, 0, 0))],
        out_shape=[jax.ShapeDtypeStruct((H, T, DH), jnp.float32)] * 3 +
                  [jax.ShapeDtypeStruct((H, NB, T), jnp.float32)],
    )(xn, wq3, bq3, wk3, bk3, wv3, bv3)
    amask4 = amask.reshape(H, NB // GB, GB, T)

    ctx = pl.pallas_call(
        _attn_kernel,
        grid=(H, T // QT),
        in_specs=[
            pl.BlockSpec((1, QT, DH), lambda h, i: (h, i, 0)),
            pl.BlockSpec((1, T, DH), lambda h, i: (h, 0, 0)),
            pl.BlockSpec((1, T, DH), lambda h, i: (h, 0, 0)),
            pl.BlockSpec((1, 1, GB, T), lambda h, i: (h, i, 0, 0)),
        ],
        out_specs=pl.BlockSpec((1, QT, DH), lambda h, i: (h, i, 0)),
        out_shape=jax.ShapeDtypeStruct((H, T, DH), jnp.float32),
    )(qh, kh, vh, amask4)
    ctx2 = ctx.transpose(1, 0, 2).reshape(T, D)

    rows = pl.BlockSpec((ROWS, D), lambda r: (r, 0))
    out = pl.pallas_call(
        _outproj_kernel,
        grid=(T // ROWS,),
        in_specs=[rows, pl.BlockSpec((D, D), lambda r: (0, 0)),
                  pl.BlockSpec((1, D), lambda r: (0, 0))],
        out_specs=rows,
        out_shape=jax.ShapeDtypeStruct((T, D), jnp.float32),
    )(ctx2, Wo, bo2)

    return out.reshape(1, T, D)


# speedup vs baseline: 8.3032x; 1.1192x over previous
import math

import jax
import jax.numpy as jnp
from jax.experimental import pallas as pl
from jax.experimental.pallas import tpu as pltpu

D = 1024
H = 16
T = 2048
BLK = 64
NB = T // BLK
KC = int(math.ceil(0.5 * NB))
DH = D // H
SCALE = 1.0 / math.sqrt(DH)
ROWS = 256
QT = 1024
GB = QT // BLK

_BF = jnp.bfloat16


def _qkv_sel_kernel(xn_ref, wq_ref, bq_ref, wk_ref, bk_ref,
                    wv_ref, bv_ref, q_ref, k_ref, v_ref, mask_ref):
    xn = xn_ref[...]
    w = jnp.concatenate([wq_ref[0], wk_ref[0], wv_ref[0]],
                        axis=0).astype(_BF)
    qkv = jax.lax.dot_general(xn, w, (((1,), (1,)), ((), ())),
                              preferred_element_type=jnp.float32)
    qh = qkv[:, :DH] + bq_ref[0]
    kh = qkv[:, DH:2 * DH] + bk_ref[0]
    vh = qkv[:, 2 * DH:] + bv_ref[0]
    q_ref[0] = qh
    k_ref[0] = kh
    v_ref[0] = vh
    qm = jnp.mean(qh.reshape(NB, BLK, DH), axis=1)
    km = jnp.mean(kh.reshape(NB, BLK, DH), axis=1)
    sim = jax.lax.dot_general(qm.astype(_BF), km.astype(_BF),
                              (((1,), (1,)), ((), ())),
                              preferred_element_type=jnp.float32)
    gt = (sim[:, None, :] > sim[:, :, None]).astype(jnp.float32)
    cntg = jnp.sum(gt, axis=-1)
    ii = jax.lax.broadcasted_iota(jnp.int32, (NB, NB), 0)
    jj = jax.lax.broadcasted_iota(jnp.int32, (NB, NB), 1)
    keep = (cntg < float(KC)) | (jj == ii)
    drop = 1.0 - keep.astype(jnp.float32)
    j_io = jax.lax.broadcasted_iota(jnp.int32, (NB, T), 0)
    c_io = jax.lax.broadcasted_iota(jnp.int32, (NB, T), 1)
    expander = (c_io // BLK == j_io).astype(_BF)
    mask_ref[0] = jax.lax.dot_general(
        (drop * (-1e30)).astype(_BF), expander, (((1,), (0,)), ((), ())),
        preferred_element_type=jnp.float32)


def _attn_kernel(q_ref, k_ref, v_ref, mask_ref, o_ref):
    i = pl.program_id(1)
    q = (q_ref[0] * SCALE).astype(_BF)

    def _tile(g, causal_add):
        kb = k_ref[0, pl.ds(g * QT, QT), :].astype(_BF)
        vb = v_ref[0, pl.ds(g * QT, QT), :].astype(_BF)
        sc = jax.lax.dot_general(q, kb, (((1,), (1,)), ((), ())),
                                 preferred_element_type=jnp.float32)
        m4 = mask_ref[0, 0, :, pl.ds(g * QT, QT)]
        sc = (sc.reshape(GB, BLK, QT) + m4[:, None, :]).reshape(QT, QT)
        if causal_add is not None:
            sc = sc + causal_add
        return sc, vb

    def _update(sc, vb, m, l, acc):
        mnew = jnp.maximum(m, jnp.max(sc, axis=1, keepdims=True))
        alpha = jnp.exp(m - mnew)
        p = jnp.exp(sc - mnew)
        l2 = l * alpha + jnp.sum(p, axis=1, keepdims=True)
        acc2 = acc * alpha + jax.lax.dot_general(
            p.astype(_BF), vb, (((1,), (0,)), ((), ())),
            preferred_element_type=jnp.float32)
        return mnew, l2, acc2

    def body(g, carry):
        m, l, acc = carry
        sc, vb = _tile(g, None)
        return _update(sc, vb, m, l, acc)

    m0 = jnp.full((QT, 1), -1e30, jnp.float32)
    l0 = jnp.zeros((QT, 1), jnp.float32)
    a0 = jnp.zeros((QT, DH), jnp.float32)
    m, l, acc = jax.lax.fori_loop(0, i, body, (m0, l0, a0))
    row = jax.lax.broadcasted_iota(jnp.int32, (QT, QT), 0)
    col = jax.lax.broadcasted_iota(jnp.int32, (QT, QT), 1)
    causal_add = jnp.where(row >= col, 0.0, -1e30)
    sc, vb = _tile(i, causal_add)
    m, l, acc = _update(sc, vb, m, l, acc)
    o_ref[0] = acc / l


def _outproj_kernel(c_ref, wo_ref, bo_ref, o_ref):
    o_ref[...] = jax.lax.dot_general(
        c_ref[...].astype(_BF), wo_ref[...].astype(_BF),
        (((1,), (1,)), ((), ())),
        preferred_element_type=jnp.float32) + bo_ref[...]


def kernel(inputs, ln_g, ln_b, Wq, bq, Wk, bk, Wv, bv, Wo, bo):
    x = inputs.reshape(T, D)
    mu = jnp.mean(x, axis=-1, keepdims=True)
    var = jnp.mean((x - mu) ** 2, axis=-1, keepdims=True)
    xn = ((x - mu) / jnp.sqrt(var + 1e-5) * ln_g.reshape(1, D)
          + ln_b.reshape(1, D)).astype(_BF)
    wq3 = Wq.reshape(H, DH, D)
    wk3 = Wk.reshape(H, DH, D)
    wv3 = Wv.reshape(H, DH, D)
    bq3 = bq.reshape(H, 1, DH)
    bk3 = bk.reshape(H, 1, DH)
    bv3 = bv.reshape(H, 1, DH)
    bo2 = bo.reshape(1, D)

    fullx = pl.BlockSpec((T, D), lambda h: (0, 0))
    whead = pl.BlockSpec((1, DH, D), lambda h: (h, 0, 0))
    bhead = pl.BlockSpec((1, 1, DH), lambda h: (h, 0, 0))
    ohead = pl.BlockSpec((1, T, DH), lambda h: (h, 0, 0))
    qh, kh, vh, amask = pl.pallas_call(
        _qkv_sel_kernel,
        grid=(H,),
        in_specs=[fullx, whead, bhead, whead, bhead, whead, bhead],
        out_specs=[ohead, ohead, ohead,
                   pl.BlockSpec((1, NB, T), lambda h: (h, 0, 0))],
        out_shape=[jax.ShapeDtypeStruct((H, T, DH), jnp.float32)] * 3 +
                  [jax.ShapeDtypeStruct((H, NB, T), jnp.float32)],
    )(xn, wq3, bq3, wk3, bk3, wv3, bv3)
    amask4 = amask.reshape(H, NB // GB, GB, T)

    ctx = pl.pallas_call(
        _attn_kernel,
        grid=(H, T // QT),
        in_specs=[
            pl.BlockSpec((1, QT, DH), lambda h, i: (h, i, 0)),
            pl.BlockSpec((1, T, DH), lambda h, i: (h, 0, 0)),
            pl.BlockSpec((1, T, DH), lambda h, i: (h, 0, 0)),
            pl.BlockSpec((1, 1, GB, T), lambda h, i: (h, i, 0, 0)),
        ],
        out_specs=pl.BlockSpec((1, QT, DH), lambda h, i: (h, i, 0)),
        out_shape=jax.ShapeDtypeStruct((H, T, DH), jnp.float32),
    )(qh, kh, vh, amask4)
    ctx2 = ctx.transpose(1, 0, 2).reshape(T, D)

    rows = pl.BlockSpec((ROWS, D), lambda r: (r, 0))
    out = pl.pallas_call(
        _outproj_kernel,
        grid=(T // ROWS,),
        in_specs=[rows, pl.BlockSpec((D, D), lambda r: (0, 0)),
                  pl.BlockSpec((1, D), lambda r: (0, 0))],
        out_specs=rows,
        out_shape=jax.ShapeDtypeStruct((T, D), jnp.float32),
    )(ctx2, Wo, bo2)

    return out.reshape(1, T, D)


# speedup vs baseline: 8.7325x; 1.0517x over previous
import math

import jax
import jax.numpy as jnp
from jax.experimental import pallas as pl
from jax.experimental.pallas import tpu as pltpu

D = 1024
H = 16
T = 2048
BLK = 64
NB = T // BLK
KC = int(math.ceil(0.5 * NB))
DH = D // H
SCALE = 1.0 / math.sqrt(DH)
ROWS = 256
QT = 1024
GB = QT // BLK

_BF = jnp.bfloat16


HG = 4


def _qkv_sel_kernel(xn_ref, wq_ref, bq_ref, wk_ref, bk_ref,
                    wv_ref, bv_ref, q_ref, k_ref, v_ref, mask_ref):
    xn = xn_ref[...]
    w = jnp.concatenate([wq_ref[...].reshape(HG * DH, D),
                         wk_ref[...].reshape(HG * DH, D),
                         wv_ref[...].reshape(HG * DH, D)],
                        axis=0).astype(_BF)
    qkv = jax.lax.dot_general(xn, w, (((1,), (1,)), ((), ())),
                              preferred_element_type=jnp.float32)
    for j in range(HG):
        qh = qkv[:, j * DH:(j + 1) * DH] + bq_ref[0, :, j * DH:(j + 1) * DH]
        kh = (qkv[:, (HG + j) * DH:(HG + j + 1) * DH]
              + bk_ref[0, :, j * DH:(j + 1) * DH])
        vh = (qkv[:, (2 * HG + j) * DH:(2 * HG + j + 1) * DH]
              + bv_ref[0, :, j * DH:(j + 1) * DH])
        q_ref[j] = qh
        k_ref[j] = kh
        v_ref[j] = vh
        qm = jnp.mean(qh.reshape(NB, BLK, DH), axis=1)
        km = jnp.mean(kh.reshape(NB, BLK, DH), axis=1)
        sim = jax.lax.dot_general(qm.astype(_BF), km.astype(_BF),
                                  (((1,), (1,)), ((), ())),
                                  preferred_element_type=jnp.float32)
        gt = (sim[:, None, :] > sim[:, :, None]).astype(jnp.float32)
        cntg = jnp.sum(gt, axis=-1)
        ii = jax.lax.broadcasted_iota(jnp.int32, (NB, NB), 0)
        jj = jax.lax.broadcasted_iota(jnp.int32, (NB, NB), 1)
        keep = (cntg < float(KC)) | (jj == ii)
        drop = 1.0 - keep.astype(jnp.float32)
        j_io = jax.lax.broadcasted_iota(jnp.int32, (NB, T), 0)
        c_io = jax.lax.broadcasted_iota(jnp.int32, (NB, T), 1)
        expander = (c_io // BLK == j_io).astype(_BF)
        mask_ref[j] = jax.lax.dot_general(
            (drop * (-1e30)).astype(_BF), expander, (((1,), (0,)), ((), ())),
            preferred_element_type=jnp.float32)


def _attn_kernel(q_ref, k_ref, v_ref, mask_ref, o_ref):
    i = pl.program_id(1)
    q = (q_ref[0] * SCALE).astype(_BF)

    def _tile(g, causal_add):
        kb = k_ref[0, pl.ds(g * QT, QT), :].astype(_BF)
        vb = v_ref[0, pl.ds(g * QT, QT), :].astype(_BF)
        sc = jax.lax.dot_general(q, kb, (((1,), (1,)), ((), ())),
                                 preferred_element_type=jnp.float32)
        m4 = mask_ref[0, 0, :, pl.ds(g * QT, QT)]
        sc = (sc.reshape(GB, BLK, QT) + m4[:, None, :]).reshape(QT, QT)
        if causal_add is not None:
            sc = sc + causal_add
        return sc, vb

    def _update(sc, vb, m, l, acc):
        mnew = jnp.maximum(m, jnp.max(sc, axis=1, keepdims=True))
        alpha = jnp.exp(m - mnew)
        p = jnp.exp(sc - mnew)
        l2 = l * alpha + jnp.sum(p, axis=1, keepdims=True)
        acc2 = acc * alpha + jax.lax.dot_general(
            p.astype(_BF), vb, (((1,), (0,)), ((), ())),
            preferred_element_type=jnp.float32)
        return mnew, l2, acc2

    def body(g, carry):
        m, l, acc = carry
        sc, vb = _tile(g, None)
        return _update(sc, vb, m, l, acc)

    m0 = jnp.full((QT, 1), -1e30, jnp.float32)
    l0 = jnp.zeros((QT, 1), jnp.float32)
    a0 = jnp.zeros((QT, DH), jnp.float32)
    m, l, acc = jax.lax.fori_loop(0, i, body, (m0, l0, a0))
    row = jax.lax.broadcasted_iota(jnp.int32, (QT, QT), 0)
    col = jax.lax.broadcasted_iota(jnp.int32, (QT, QT), 1)
    causal_add = jnp.where(row >= col, 0.0, -1e30)
    sc, vb = _tile(i, causal_add)
    m, l, acc = _update(sc, vb, m, l, acc)
    o_ref[0] = acc / l


def _outproj_kernel(c_ref, wo_ref, bo_ref, o_ref):
    o_ref[...] = jax.lax.dot_general(
        c_ref[...].astype(_BF), wo_ref[...].astype(_BF),
        (((1,), (1,)), ((), ())),
        preferred_element_type=jnp.float32) + bo_ref[...]


def kernel(inputs, ln_g, ln_b, Wq, bq, Wk, bk, Wv, bv, Wo, bo):
    x = inputs.reshape(T, D)
    mu = jnp.mean(x, axis=-1, keepdims=True)
    var = jnp.mean((x - mu) ** 2, axis=-1, keepdims=True)
    xn = ((x - mu) / jnp.sqrt(var + 1e-5) * ln_g.reshape(1, D)
          + ln_b.reshape(1, D)).astype(_BF)
    wq3 = Wq.reshape(H, DH, D)
    wk3 = Wk.reshape(H, DH, D)
    wv3 = Wv.reshape(H, DH, D)
    bq3 = bq.reshape(H // HG, 1, HG * DH)
    bk3 = bk.reshape(H // HG, 1, HG * DH)
    bv3 = bv.reshape(H // HG, 1, HG * DH)
    bo2 = bo.reshape(1, D)

    fullx = pl.BlockSpec((T, D), lambda h: (0, 0))
    whead = pl.BlockSpec((HG, DH, D), lambda h: (h, 0, 0))
    bhead = pl.BlockSpec((1, 1, HG * DH), lambda h: (h, 0, 0))
    ohead = pl.BlockSpec((HG, T, DH), lambda h: (h, 0, 0))
    qh, kh, vh, amask = pl.pallas_call(
        _qkv_sel_kernel,
        grid=(H // HG,),
        in_specs=[fullx, whead, bhead, whead, bhead, whead, bhead],
        out_specs=[ohead, ohead, ohead,
                   pl.BlockSpec((HG, NB, T), lambda h: (h, 0, 0))],
        out_shape=[jax.ShapeDtypeStruct((H, T, DH), jnp.float32)] * 3 +
                  [jax.ShapeDtypeStruct((H, NB, T), jnp.float32)],
    )(xn, wq3, bq3, wk3, bk3, wv3, bv3)
    amask4 = amask.reshape(H, NB // GB, GB, T)

    ctx = pl.pallas_call(
        _attn_kernel,
        grid=(H, T // QT),
        in_specs=[
            pl.BlockSpec((1, QT, DH), lambda h, i: (h, i, 0)),
            pl.BlockSpec((1, T, DH), lambda h, i: (h, 0, 0)),
            pl.BlockSpec((1, T, DH), lambda h, i: (h, 0, 0)),
            pl.BlockSpec((1, 1, GB, T), lambda h, i: (h, i, 0, 0)),
        ],
        out_specs=pl.BlockSpec((1, QT, DH), lambda h, i: (h, i, 0)),
        out_shape=jax.ShapeDtypeStruct((H, T, DH), jnp.float32),
    )(qh, kh, vh, amask4)
    ctx2 = ctx.transpose(1, 0, 2).reshape(T, D)

    rows = pl.BlockSpec((ROWS, D), lambda r: (r, 0))
    out = pl.pallas_call(
        _outproj_kernel,
        grid=(T // ROWS,),
        in_specs=[rows, pl.BlockSpec((D, D), lambda r: (0, 0)),
                  pl.BlockSpec((1, D), lambda r: (0, 0))],
        out_specs=rows,
        out_shape=jax.ShapeDtypeStruct((T, D), jnp.float32),
    )(ctx2, Wo, bo2)

    return out.reshape(1, T, D)
